# baseline (speedup 1.0000x reference)
import functools

import jax
import jax.numpy as jnp
from jax import lax
from jax.experimental import pallas as pl
from jax.experimental.pallas import tpu as pltpu
from jax.experimental.pallas import tpu_sc as plsc

N = 10000
E = 320000
NC = 2
NS = 16
NW = NC * NS
EPW = E // NW
K = 400
NCHUNK = EPW // K
RPS = N // NS
ZR = 125
HW = 16

_mesh = plsc.VectorSubcoreMesh(core_axis_name="c", subcore_axis_name="s")
_sc_params = pltpu.CompilerParams(use_tc_tiling_on_sc=False)
_sc_hist_params = pltpu.CompilerParams(use_tc_tiling_on_sc=False,
                                       needs_layout_passes=False)
NR = N // 16


@functools.partial(
    pl.kernel,
    out_type=jax.ShapeDtypeStruct((NC, N, HW), jnp.float32),
    mesh=_mesh,
    compiler_params=_sc_params,
    scratch_types=[
        pltpu.VMEM((NCHUNK, K), jnp.int32),
        pltpu.VMEM((K, HW), jnp.float32),
        pltpu.VMEM((ZR, HW), jnp.float32),
        pltpu.VMEM_SHARED((N, HW), jnp.float32),
        pltpu.SemaphoreType.DMA,
    ],
)
def _sc_hist(ei_hbm, out_hbm, dst_v, ones_v, zbuf, acc, sem):
    cid = lax.axis_index("c")
    sid = lax.axis_index("s")
    wid = sid * NC + cid

    @pl.loop(0, ZR)
    def _(r):
        zbuf[r, :] = jnp.zeros((HW,), jnp.float32)

    @pl.loop(0, K)
    def _(r):
        ones_v[r, :] = jnp.ones((HW,), jnp.float32)

    @pl.loop(0, RPS // ZR)
    def _(r):
        pltpu.sync_copy(zbuf, acc.at[pl.ds(sid * RPS + r * ZR, ZR)])

    @pl.loop(0, NCHUNK)
    def _(c):
        pltpu.async_copy(ei_hbm.at[1, pl.ds(wid * EPW + c * K, K)],
                         dst_v.at[c], sem)

    @pl.loop(0, NCHUNK)
    def _(c):
        pltpu.make_async_copy(ei_hbm.at[1, pl.ds(0, K)], dst_v.at[0],
                              sem).wait()

    plsc.subcore_barrier()

    @pl.loop(0, NCHUNK)
    def _(ci):
        pltpu.sync_copy(ones_v, acc.at[dst_v.at[ci]], add=True)

    plsc.subcore_barrier()
    pltpu.sync_copy(
        acc.at[pl.ds(sid * RPS, RPS)],
        out_hbm.at[cid, pl.ds(sid * RPS, RPS)],
    )


def _make_sc_agg(D):
    @functools.partial(
        pl.kernel,
        out_type=jax.ShapeDtypeStruct((NC, N, D), jnp.float32),
        mesh=_mesh,
        compiler_params=_sc_params,
        scratch_types=[
            pltpu.VMEM((NCHUNK, K), jnp.int32),
            pltpu.VMEM((NCHUNK, K), jnp.int32),
            pltpu.VMEM((K, D), jnp.float32),
            pltpu.VMEM((K, D), jnp.float32),
            pltpu.VMEM((ZR, D), jnp.float32),
            pltpu.VMEM_SHARED((N, D), jnp.float32),
            pltpu.SemaphoreType.DMA,
            pltpu.SemaphoreType.DMA,
            pltpu.SemaphoreType.DMA,
            pltpu.SemaphoreType.DMA,
            pltpu.SemaphoreType.DMA,
        ],
    )
    def _sc_agg(h_hbm, ei_hbm, out_hbm,
                src_v, dst_v, buf0, buf1, zbuf, acc,
                sem0, sem1, sems0, sems1, semi):
        cid = lax.axis_index("c")
        sid = lax.axis_index("s")
        wid = sid * NC + cid

        @pl.loop(0, ZR)
        def _(r):
            @pl.loop(0, D // 16)
            def _(j):
                zbuf[r, pl.ds(j * 16, 16)] = jnp.zeros((16,), jnp.float32)

        @pl.loop(0, RPS // ZR)
        def _(r):
            pltpu.sync_copy(zbuf, acc.at[pl.ds(sid * RPS + r * ZR, ZR)])

        @pl.loop(0, NCHUNK)
        def _(c):
            base = wid * EPW + c * K
            pltpu.async_copy(ei_hbm.at[0, pl.ds(base, K)], src_v.at[c], semi)
            pltpu.async_copy(ei_hbm.at[1, pl.ds(base, K)], dst_v.at[c], semi)

        @pl.loop(0, 2 * NCHUNK)
        def _(c):
            pltpu.make_async_copy(ei_hbm.at[0, pl.ds(0, K)], src_v.at[0],
                                  semi).wait()

        plsc.subcore_barrier()

        def g_start(c, b, g):
            pltpu.async_copy(h_hbm.at[src_v.at[c]], b, g)

        def g_wait(c, b, g):
            pltpu.make_async_copy(h_hbm.at[src_v.at[c]], b, g).wait()

        def s_start(c, b, sm):
            pltpu.async_copy(b, acc.at[dst_v.at[c]], sm, add=True)

        def s_wait(c, b, sm):
            pltpu.make_async_copy(b, acc.at[dst_v.at[c]], sm).wait()

        g_start(0, buf0, sem0)
        g_start(1, buf1, sem1)

        @pl.loop(0, (NCHUNK - 1) // 2)
        def _(i):
            c = 2 * i
            g_wait(c, buf0, sem0)
            s_start(c, buf0, sems0)
            g_wait(c + 1, buf1, sem1)
            s_start(c + 1, buf1, sems1)
            s_wait(c, buf0, sems0)
            g_start(c + 2, buf0, sem0)
            s_wait(c + 1, buf1, sems1)

            @pl.when(i < (NCHUNK - 1) // 2 - 1)
            def _():
                g_start(c + 3, buf1, sem1)

        g_wait(NCHUNK - 1, buf0, sem0)
        pltpu.sync_copy(buf0, acc.at[dst_v.at[NCHUNK - 1]], add=True)

        plsc.subcore_barrier()
        pltpu.sync_copy(
            acc.at[pl.ds(sid * RPS, RPS)],
            out_hbm.at[cid, pl.ds(sid * RPS, RPS)],
        )

    return _sc_agg


_sc_agg64 = _make_sc_agg(64)
_sc_agg32 = _make_sc_agg(32)


_RB = 1000


def _mm_body(x_ref, w_ref, o_ref):
    o_ref[...] = jnp.dot(x_ref[...], w_ref[...],
                         preferred_element_type=jnp.float32)


def _tc_matmul(x, w):
    n, d_in = x.shape
    d_out = w.shape[1]
    return pl.pallas_call(
        _mm_body,
        grid=(n // _RB,),
        in_specs=[
            pl.BlockSpec((_RB, d_in), lambda i: (i, 0)),
            pl.BlockSpec((d_in, d_out), lambda i: (0, 0)),
        ],
        out_specs=pl.BlockSpec((_RB, d_out), lambda i: (i, 0)),
        out_shape=jax.ShapeDtypeStruct((n, d_out), jnp.float32),
    )(x, w)


def _scale_body(hp0_ref, hp1_ref, h_ref, hs_ref, dis_ref):
    deg = hp0_ref[...] + hp1_ref[...] + 1.0
    dis = lax.rsqrt(deg)
    dis_ref[...] = dis
    hs_ref[...] = dis * h_ref[...]


def _tc_scale(hp0, hp1, h):
    d = h.shape[1]
    return pl.pallas_call(
        _scale_body,
        grid=(N // _RB,),
        in_specs=[
            pl.BlockSpec((_RB, 1), lambda i: (i, 0)),
            pl.BlockSpec((_RB, 1), lambda i: (i, 0)),
            pl.BlockSpec((_RB, d), lambda i: (i, 0)),
        ],
        out_specs=[
            pl.BlockSpec((_RB, d), lambda i: (i, 0)),
            pl.BlockSpec((_RB, 1), lambda i: (i, 0)),
        ],
        out_shape=[
            jax.ShapeDtypeStruct((N, d), jnp.float32),
            jax.ShapeDtypeStruct((N, 1), jnp.float32),
        ],
    )(hp0, hp1, h)


def _layer_body(dis_ref, a_ref, hs_ref, b_ref, w_ref, o_ref):
    dis = dis_ref[...]
    z = dis * (a_ref[0] + a_ref[1] + hs_ref[...]) + b_ref[...]
    z = jnp.maximum(z, 0.0)
    h = jnp.dot(z, w_ref[...], preferred_element_type=jnp.float32)
    o_ref[...] = dis * h


def _tc_layer(dis, a, hs, b, w):
    d = hs.shape[1]
    d_out = w.shape[1]
    return pl.pallas_call(
        _layer_body,
        grid=(N // _RB,),
        in_specs=[
            pl.BlockSpec((_RB, 1), lambda i: (i, 0)),
            pl.BlockSpec((NC, _RB, d), lambda i: (0, i, 0)),
            pl.BlockSpec((_RB, d), lambda i: (i, 0)),
            pl.BlockSpec((1, d), lambda i: (0, 0)),
            pl.BlockSpec((d, d_out), lambda i: (0, 0)),
        ],
        out_specs=pl.BlockSpec((_RB, d_out), lambda i: (i, 0)),
        out_shape=jax.ShapeDtypeStruct((N, d_out), jnp.float32),
    )(dis, a, hs, b, w)


def _head_body(dis_ref, a_ref, hs_ref, b_ref, w_ref, bf_ref, o_ref):
    dis = dis_ref[...]
    z = dis * (a_ref[0] + a_ref[1] + hs_ref[...]) + b_ref[...]
    z = jnp.maximum(z, 0.0)
    o = jnp.dot(z, w_ref[...], preferred_element_type=jnp.float32)
    o_ref[...] = jax.nn.sigmoid(o + bf_ref[...])


def _tc_head(dis, a, hs, b, w, bf):
    d = hs.shape[1]
    return pl.pallas_call(
        _head_body,
        grid=(N // _RB,),
        in_specs=[
            pl.BlockSpec((_RB, 1), lambda i: (i, 0)),
            pl.BlockSpec((NC, _RB, d), lambda i: (0, i, 0)),
            pl.BlockSpec((_RB, d), lambda i: (i, 0)),
            pl.BlockSpec((1, d), lambda i: (0, 0)),
            pl.BlockSpec((d, 1), lambda i: (0, 0)),
            pl.BlockSpec((1, 1), lambda i: (0, 0)),
        ],
        out_specs=pl.BlockSpec((_RB, 1), lambda i: (i, 0)),
        out_shape=jax.ShapeDtypeStruct((N, 1), jnp.float32),
    )(dis, a, hs, b, w, bf)


def kernel(x, edge_index, W1, b1, W2, b2, Wfc, bfc):
    hp = _sc_hist(edge_index)
    h1 = _tc_matmul(x, W1)
    h1s, dis = _tc_scale(hp[0, :, 0:1], hp[1, :, 0:1], h1)

    ap = _sc_agg64(h1s, edge_index)
    h2s = _tc_layer(dis, ap, h1s, b1.reshape(1, 64), W2)

    bp = _sc_agg32(h2s, edge_index)
    o = _tc_head(dis, bp, h2s, b2.reshape(1, 32), Wfc, bfc.reshape(1, 1))
    return o[:, 0]


# speedup vs baseline: 1.1654x; 1.1654x over previous
import functools

import jax
import jax.numpy as jnp
from jax import lax
from jax.experimental import pallas as pl
from jax.experimental.pallas import tpu as pltpu
from jax.experimental.pallas import tpu_sc as plsc

N = 10000
E = 320000
NC = 2
NS = 16
NW = NC * NS
EPW = E // NW
K = 400
NCHUNK = EPW // K
RPS = N // NS
ZR = 125
HW = 16

_mesh = plsc.VectorSubcoreMesh(core_axis_name="c", subcore_axis_name="s")
_sc_params = pltpu.CompilerParams(use_tc_tiling_on_sc=False)
_sc_hist_params = pltpu.CompilerParams(use_tc_tiling_on_sc=False,
                                       needs_layout_passes=False)
NR = N // 16


@functools.partial(
    pl.kernel,
    out_type=jax.ShapeDtypeStruct((NC, N, HW), jnp.float32),
    mesh=_mesh,
    compiler_params=_sc_params,
    scratch_types=[
        pltpu.VMEM((NCHUNK, K), jnp.int32),
        pltpu.VMEM((K, HW), jnp.float32),
        pltpu.VMEM((ZR, HW), jnp.float32),
        pltpu.VMEM_SHARED((N, HW), jnp.float32),
        pltpu.SemaphoreType.DMA,
    ],
)
def _sc_hist(ei_hbm, out_hbm, dst_v, ones_v, zbuf, acc, sem):
    cid = lax.axis_index("c")
    sid = lax.axis_index("s")
    wid = sid * NC + cid

    @pl.loop(0, ZR)
    def _(r):
        zbuf[r, :] = jnp.zeros((HW,), jnp.float32)

    @pl.loop(0, K)
    def _(r):
        ones_v[r, :] = jnp.ones((HW,), jnp.float32)

    @pl.loop(0, RPS // ZR)
    def _(r):
        pltpu.sync_copy(zbuf, acc.at[pl.ds(sid * RPS + r * ZR, ZR)])

    @pl.loop(0, NCHUNK)
    def _(c):
        pltpu.async_copy(ei_hbm.at[1, pl.ds(wid * EPW + c * K, K)],
                         dst_v.at[c], sem)

    @pl.loop(0, NCHUNK)
    def _(c):
        pltpu.make_async_copy(ei_hbm.at[1, pl.ds(0, K)], dst_v.at[0],
                              sem).wait()

    plsc.subcore_barrier()

    @pl.loop(0, NCHUNK)
    def _(ci):
        pltpu.sync_copy(ones_v, acc.at[dst_v.at[ci]], add=True)

    plsc.subcore_barrier()
    pltpu.sync_copy(
        acc.at[pl.ds(sid * RPS, RPS)],
        out_hbm.at[cid, pl.ds(sid * RPS, RPS)],
    )


def _make_sc_agg(D):
    @functools.partial(
        pl.kernel,
        out_type=jax.ShapeDtypeStruct((NC, N, D), jnp.float32),
        mesh=_mesh,
        compiler_params=_sc_params,
        scratch_types=[
            pltpu.VMEM((NCHUNK, K), jnp.int32),
            pltpu.VMEM((NCHUNK, K), jnp.int32),
            pltpu.VMEM((K, D), jnp.float32),
            pltpu.VMEM((K, D), jnp.float32),
            pltpu.VMEM((ZR, D), jnp.float32),
            pltpu.VMEM_SHARED((N, D), jnp.float32),
            pltpu.SemaphoreType.DMA,
            pltpu.SemaphoreType.DMA,
            pltpu.SemaphoreType.DMA,
        ],
    )
    def _sc_agg(h_hbm, ei_hbm, out_hbm,
                src_v, dst_v, buf0, buf1, zbuf, acc, sem0, sem1, semi):
        cid = lax.axis_index("c")
        sid = lax.axis_index("s")
        wid = sid * NC + cid

        @pl.loop(0, ZR)
        def _(r):
            @pl.loop(0, D // 16)
            def _(j):
                zbuf[r, pl.ds(j * 16, 16)] = jnp.zeros((16,), jnp.float32)

        @pl.loop(0, RPS // ZR)
        def _(r):
            pltpu.sync_copy(zbuf, acc.at[pl.ds(sid * RPS + r * ZR, ZR)])

        @pl.loop(0, NCHUNK)
        def _(c):
            base = wid * EPW + c * K
            pltpu.async_copy(ei_hbm.at[0, pl.ds(base, K)], src_v.at[c], semi)
            pltpu.async_copy(ei_hbm.at[1, pl.ds(base, K)], dst_v.at[c], semi)

        @pl.loop(0, 2 * NCHUNK)
        def _(c):
            pltpu.make_async_copy(ei_hbm.at[0, pl.ds(0, K)], src_v.at[0],
                                  semi).wait()

        plsc.subcore_barrier()

        pltpu.async_copy(h_hbm.at[src_v.at[0]], buf0, sem0)

        @pl.loop(0, (NCHUNK - 1) // 2)
        def _(i):
            c = 2 * i
            pltpu.async_copy(h_hbm.at[src_v.at[c + 1]], buf1, sem1)
            pltpu.make_async_copy(h_hbm.at[src_v.at[c]], buf0, sem0).wait()
            pltpu.sync_copy(buf0, acc.at[dst_v.at[c]], add=True)
            pltpu.async_copy(h_hbm.at[src_v.at[c + 2]], buf0, sem0)
            pltpu.make_async_copy(h_hbm.at[src_v.at[c + 1]], buf1, sem1).wait()
            pltpu.sync_copy(buf1, acc.at[dst_v.at[c + 1]], add=True)

        pltpu.make_async_copy(h_hbm.at[src_v.at[NCHUNK - 1]], buf0, sem0).wait()
        pltpu.sync_copy(buf0, acc.at[dst_v.at[NCHUNK - 1]], add=True)

        plsc.subcore_barrier()
        pltpu.sync_copy(
            acc.at[pl.ds(sid * RPS, RPS)],
            out_hbm.at[cid, pl.ds(sid * RPS, RPS)],
        )

    return _sc_agg


_sc_agg64 = _make_sc_agg(64)
_sc_agg32 = _make_sc_agg(32)


_RB = 1000


def _mm_body(x_ref, w_ref, o_ref):
    o_ref[...] = jnp.dot(x_ref[...], w_ref[...],
                         preferred_element_type=jnp.float32)


def _tc_matmul(x, w):
    n, d_in = x.shape
    d_out = w.shape[1]
    return pl.pallas_call(
        _mm_body,
        grid=(n // _RB,),
        in_specs=[
            pl.BlockSpec((_RB, d_in), lambda i: (i, 0)),
            pl.BlockSpec((d_in, d_out), lambda i: (0, 0)),
        ],
        out_specs=pl.BlockSpec((_RB, d_out), lambda i: (i, 0)),
        out_shape=jax.ShapeDtypeStruct((n, d_out), jnp.float32),
    )(x, w)


def _scale_body(hp_ref, h_ref, hs_ref, dis_ref):
    deg = hp_ref[0, :, 0:1] + hp_ref[1, :, 0:1] + 1.0
    dis = lax.rsqrt(deg)
    dis_ref[...] = dis
    hs_ref[...] = dis * h_ref[...]


def _tc_scale(hp, h):
    d = h.shape[1]
    return pl.pallas_call(
        _scale_body,
        grid=(N // _RB,),
        in_specs=[
            pl.BlockSpec((NC, _RB, HW), lambda i: (0, i, 0)),
            pl.BlockSpec((_RB, d), lambda i: (i, 0)),
        ],
        out_specs=[
            pl.BlockSpec((_RB, d), lambda i: (i, 0)),
            pl.BlockSpec((_RB, 1), lambda i: (i, 0)),
        ],
        out_shape=[
            jax.ShapeDtypeStruct((N, d), jnp.float32),
            jax.ShapeDtypeStruct((N, 1), jnp.float32),
        ],
    )(hp, h)


def _layer_body(dis_ref, a_ref, hs_ref, b_ref, w_ref, o_ref):
    dis = dis_ref[...]
    z = dis * (a_ref[0] + a_ref[1] + hs_ref[...]) + b_ref[...]
    z = jnp.maximum(z, 0.0)
    h = jnp.dot(z, w_ref[...], preferred_element_type=jnp.float32)
    o_ref[...] = dis * h


def _tc_layer(dis, a, hs, b, w):
    d = hs.shape[1]
    d_out = w.shape[1]
    return pl.pallas_call(
        _layer_body,
        grid=(N // _RB,),
        in_specs=[
            pl.BlockSpec((_RB, 1), lambda i: (i, 0)),
            pl.BlockSpec((NC, _RB, d), lambda i: (0, i, 0)),
            pl.BlockSpec((_RB, d), lambda i: (i, 0)),
            pl.BlockSpec((1, d), lambda i: (0, 0)),
            pl.BlockSpec((d, d_out), lambda i: (0, 0)),
        ],
        out_specs=pl.BlockSpec((_RB, d_out), lambda i: (i, 0)),
        out_shape=jax.ShapeDtypeStruct((N, d_out), jnp.float32),
    )(dis, a, hs, b, w)


def _head_body(dis_ref, a_ref, hs_ref, b_ref, w_ref, bf_ref, o_ref):
    dis = dis_ref[...]
    z = dis * (a_ref[0] + a_ref[1] + hs_ref[...]) + b_ref[...]
    z = jnp.maximum(z, 0.0)
    o = jnp.dot(z, w_ref[...], preferred_element_type=jnp.float32)
    o_ref[...] = jax.nn.sigmoid(o + bf_ref[...])


def _tc_head(dis, a, hs, b, w, bf):
    d = hs.shape[1]
    return pl.pallas_call(
        _head_body,
        grid=(N // _RB,),
        in_specs=[
            pl.BlockSpec((_RB, 1), lambda i: (i, 0)),
            pl.BlockSpec((NC, _RB, d), lambda i: (0, i, 0)),
            pl.BlockSpec((_RB, d), lambda i: (i, 0)),
            pl.BlockSpec((1, d), lambda i: (0, 0)),
            pl.BlockSpec((d, 1), lambda i: (0, 0)),
            pl.BlockSpec((1, 1), lambda i: (0, 0)),
        ],
        out_specs=pl.BlockSpec((_RB, 1), lambda i: (i, 0)),
        out_shape=jax.ShapeDtypeStruct((N, 1), jnp.float32),
    )(dis, a, hs, b, w, bf)


def kernel(x, edge_index, W1, b1, W2, b2, Wfc, bfc):
    hp = _sc_hist(edge_index)
    h1 = _tc_matmul(x, W1)
    h1s, dis = _tc_scale(hp, h1)

    ap = _sc_agg64(h1s, edge_index)
    h2s = _tc_layer(dis, ap, h1s, b1.reshape(1, 64), W2)

    bp = _sc_agg32(h2s, edge_index)
    o = _tc_head(dis, bp, h2s, b2.reshape(1, 32), Wfc, bfc.reshape(1, 1))
    return o[:, 0]
